# baseline (speedup 1.0000x reference)
import functools

import jax
import jax.numpy as jnp
from jax import lax
from jax.experimental import pallas as pl
from jax.experimental.pallas import tpu as pltpu
from jax.experimental.pallas import tpu_sc as plsc

_ALPHA = 4
_NC = 2
_NS = 16
_NCHUNK = 4


_NDMA = 8


def _tc_copy_body(x_hbm, o_hbm, *sems):
    rows = x_hbm.shape[0]
    chunk = rows // _NDMA
    copies = [
        pltpu.make_async_copy(
            x_hbm.at[pl.ds(i * chunk, chunk)],
            o_hbm.at[pl.ds(i * chunk, chunk)],
            sems[i],
        )
        for i in range(_NDMA)
    ]
    for c in copies:
        c.start()
    for c in copies:
        c.wait()


def _sc_gather_body(T, S, D, flat_hbm, out_hbm, buf0, buf1, sem0, sem1):
    c = lax.axis_index("c")
    s = lax.axis_index("s")
    w = c * _NS + s
    b = w // S
    j = w % S
    src = b * T + (j * (T - 1)) // (S - 1)

    ch = D // _NCHUNK
    bufs = (buf0, buf1)
    sems = (sem0, sem1)
    copies = [None, None]
    copies[0] = pltpu.make_async_copy(
        flat_hbm.at[src, pl.ds(0, ch)], bufs[0], sems[0])
    copies[0].start()
    for k in range(_NCHUNK):
        nk = k + 1
        if nk < _NCHUNK:
            copies[nk % 2] = pltpu.make_async_copy(
                flat_hbm.at[src, pl.ds(nk * ch, ch)], bufs[nk % 2], sems[nk % 2])
            copies[nk % 2].start()
        copies[k % 2].wait()
        pltpu.sync_copy(bufs[k % 2], out_hbm.at[w, pl.ds(k * ch, ch)])


def kernel(frames):
    B, T, C, H, W = frames.shape
    S = T // _ALPHA
    D = C * H * W
    assert B * S == _NC * _NS, "one slow frame per SC vector subcore"
    import numpy as _np
    _expect = _np.linspace(0.0, T - 1, S).astype(_np.int32)
    _got = (_np.arange(S) * (T - 1)) // (S - 1)
    assert _np.array_equal(_expect, _got), (_expect, _got)

    flat = frames.reshape(B * T, D)

    ch = D // _NCHUNK
    slow_flat = pl.kernel(
        functools.partial(_sc_gather_body, T, S, D),
        out_type=jax.ShapeDtypeStruct((B * S, D), jnp.float32),
        mesh=plsc.VectorSubcoreMesh(core_axis_name="c", subcore_axis_name="s"),
        scratch_types=[
            pltpu.VMEM((ch,), jnp.float32),
            pltpu.VMEM((ch,), jnp.float32),
            pltpu.SemaphoreType.DMA,
            pltpu.SemaphoreType.DMA,
        ],
    )(flat)
    slow = slow_flat.reshape(B, S, C, H, W)

    rows = B * T
    d2 = D // 128
    flat3 = frames.reshape(rows, d2, 128)
    fast3 = pl.pallas_call(
        _tc_copy_body,
        in_specs=[pl.BlockSpec(memory_space=pltpu.HBM)],
        out_specs=pl.BlockSpec(memory_space=pltpu.HBM),
        out_shape=jax.ShapeDtypeStruct((rows, d2, 128), jnp.float32),
        scratch_shapes=[pltpu.SemaphoreType.DMA] * _NDMA,
    )(flat3)
    fast = fast3.reshape(B, T, C, H, W)

    return (slow, fast)


# speedup vs baseline: 31.7494x; 31.7494x over previous
import functools

import jax
import jax.numpy as jnp
from jax import lax
from jax.experimental import pallas as pl
from jax.experimental.pallas import tpu as pltpu
from jax.experimental.pallas import tpu_sc as plsc

_ALPHA = 4
_NC = 2
_NS = 16


def _tc_copy_body(x_ref, o_ref):
    o_ref[...] = x_ref[...]


def _sc_gather_body(T, S, frames_hbm, out_hbm, buf0, buf1, sem0, sem1):
    c = lax.axis_index("c")
    s = lax.axis_index("s")
    w = c * _NS + s
    b = w // S
    j = w % S
    t = (j * (T - 1)) // (S - 1)

    nch = frames_hbm.shape[2]
    bufs = (buf0, buf1)
    sems = (sem0, sem1)
    copies = [None, None]
    copies[0] = pltpu.make_async_copy(frames_hbm.at[b, t, 0], bufs[0], sems[0])
    copies[0].start()
    for k in range(nch):
        nk = k + 1
        if nk < nch:
            copies[nk % 2] = pltpu.make_async_copy(
                frames_hbm.at[b, t, nk], bufs[nk % 2], sems[nk % 2])
            copies[nk % 2].start()
        copies[k % 2].wait()
        pltpu.sync_copy(bufs[k % 2], out_hbm.at[b, j, k])


def kernel(frames):
    B, T, C, H, W = frames.shape
    S = T // _ALPHA
    assert B * S == _NC * _NS, "one slow frame per SC vector subcore"
    import numpy as _np
    _expect = _np.linspace(0.0, T - 1, S).astype(_np.int32)
    _got = (_np.arange(S) * (T - 1)) // (S - 1)
    assert _np.array_equal(_expect, _got), (_expect, _got)

    slow = pl.kernel(
        functools.partial(_sc_gather_body, T, S),
        out_type=jax.ShapeDtypeStruct((B, S, C, H, W), jnp.float32),
        mesh=plsc.VectorSubcoreMesh(core_axis_name="c", subcore_axis_name="s"),
        scratch_types=[
            pltpu.VMEM((H, W), jnp.float32),
            pltpu.VMEM((H, W), jnp.float32),
            pltpu.SemaphoreType.DMA,
            pltpu.SemaphoreType.DMA,
        ],
    )(frames)

    blk_t = 8
    fast = pl.pallas_call(
        _tc_copy_body,
        grid=(B, T // blk_t),
        in_specs=[pl.BlockSpec((1, blk_t, C, H, W),
                               lambda b, i: (b, i, 0, 0, 0))],
        out_specs=pl.BlockSpec((1, blk_t, C, H, W),
                               lambda b, i: (b, i, 0, 0, 0)),
        out_shape=jax.ShapeDtypeStruct((B, T, C, H, W), jnp.float32),
    )(frames)

    return (slow, fast)
